# baseline (speedup 1.0000x reference)
import functools

import numpy as np
import jax
import jax.numpy as jnp
from jax import lax
from jax.experimental import pallas as pl
from jax.experimental.pallas import tpu as pltpu
from jax.experimental.pallas import tpu_sc as plsc

N = 10000
E = 320000
F1 = 128
F2 = 64
B = 1024
L = 50
LPAD = 64
NW = 32
EPW = E // NW
K = 80
NCH = EPW // K
RPT = 624
BPW = B // NW

_mesh = plsc.VectorSubcoreMesh(core_axis_name="c", subcore_axis_name="s",
                               num_cores=2, num_subcores=16)

_loc_np = np.zeros((16, 16, 128), np.int32)
for _s in range(16):
    for _t in range(16):
        _loc_np[_s, _t, :64] = _s * BPW + 2 * _t
        _loc_np[_s, _t, 64:] = _s * BPW + 2 * _t + 1


def _striped(s, fn):
    fn(s * RPT, RPT)

    @pl.when(s == 15)
    def _tail():
        fn(16 * RPT, N - 16 * RPT)


def _edge_body(F, with_deg, n_phase, cpp, *refs):
    if with_deg:
        (tab, src4, dst4, z2d, agg_out, deg_out,
         srcb, dstb, rows0, rows1, ones, dbuf, acc, dacc, sem0, sem1) = refs
    else:
        (tab, src4, dst4, z2d, agg_out,
         srcb, dstb, rows0, rows1, acc, sem0, sem1) = refs
    c = lax.axis_index("c")
    s = lax.axis_index("s")
    wid = c * 16 + s
    _striped(s, lambda o, n: pltpu.sync_copy(z2d.at[pl.ds(o, n), :],
                                             acc.at[pl.ds(o, n), :]))
    if with_deg:
        for i in range(K // 16):
            ones[pl.ds(i * 16, 16)] = jnp.ones((16,), jnp.float32)
        for i in range(RPT // 16):
            dbuf[pl.ds(i * 16, 16)] = jnp.zeros((16,), jnp.float32)
        _striped(s, lambda o, n: pltpu.sync_copy(dbuf.at[pl.ds(0, n)],
                                                 dacc.at[pl.ds(o, n)]))
    plsc.subcore_barrier()

    def g_start(j, buf, sem):
        pltpu.async_copy(tab.at[srcb.at[j]], buf, sem)

    def g_wait(buf, sem):
        pltpu.make_async_copy(tab.at[srcb.at[0]], buf, sem).wait()

    def scat(j, buf):
        pltpu.sync_copy(buf, acc.at[dstb.at[j]], add=True)
        if with_deg:
            pltpu.sync_copy(ones, dacc.at[dstb.at[j]], add=True)

    for p in range(n_phase):
        pltpu.sync_copy(src4.at[wid, p], srcb)
        pltpu.sync_copy(dst4.at[wid, p], dstb)
        g_start(0, rows0, sem0)

        def chunk2(j2, carry):
            a = 2 * j2
            g_wait(rows0, sem0)
            g_start(a + 1, rows1, sem1)
            scat(a, rows0)
            g_wait(rows1, sem1)
            g_start(a + 2, rows0, sem0)
            scat(a + 1, rows1)
            return carry

        lax.fori_loop(0, (cpp - 1) // 2, chunk2, 0)
        g_wait(rows0, sem0)
        scat(cpp - 1, rows0)
    plsc.subcore_barrier()
    _striped(s, lambda o, n: pltpu.sync_copy(acc.at[pl.ds(o, n), :],
                                             agg_out.at[c, pl.ds(o, n), :]))
    if with_deg:
        def _deg_out(o, n):
            pltpu.sync_copy(dacc.at[pl.ds(o, n)], dbuf.at[pl.ds(0, n)])
            pltpu.sync_copy(dbuf.at[pl.ds(0, n)], deg_out.at[pl.ds(c * N + o, n)])
        _striped(s, _deg_out)


NP1 = 5
CPP1 = NCH // NP1

_edge128 = pl.kernel(
    functools.partial(_edge_body, F1, True, NP1, CPP1),
    out_type=(jax.ShapeDtypeStruct((2, N, F1), jnp.float32),
              jax.ShapeDtypeStruct((2 * N,), jnp.float32)),
    mesh=_mesh,
    scratch_types=[
        pltpu.VMEM((CPP1, K), jnp.int32),
        pltpu.VMEM((CPP1, K), jnp.int32),
        pltpu.VMEM((K, F1), jnp.float32),
        pltpu.VMEM((K, F1), jnp.float32),
        pltpu.VMEM((K,), jnp.float32),
        pltpu.VMEM((RPT,), jnp.float32),
        pltpu.VMEM_SHARED((N, F1), jnp.float32),
        pltpu.VMEM_SHARED((N,), jnp.float32),
        pltpu.SemaphoreType.DMA,
        pltpu.SemaphoreType.DMA,
    ],
)

_edge64 = pl.kernel(
    functools.partial(_edge_body, F2, False, 1, NCH),
    out_type=jax.ShapeDtypeStruct((2, N, F2), jnp.float32),
    mesh=_mesh,
    scratch_types=[
        pltpu.VMEM((NCH, K), jnp.int32),
        pltpu.VMEM((NCH, K), jnp.int32),
        pltpu.VMEM((K, F2), jnp.float32),
        pltpu.VMEM((K, F2), jnp.float32),
        pltpu.VMEM_SHARED((N, F2), jnp.float32),
        pltpu.SemaphoreType.DMA,
        pltpu.SemaphoreType.DMA,
    ],
    compiler_params=pltpu.CompilerParams(use_tc_tiling_on_sc=False),
)


def _sent_body(x_hbm, sidx, out_hbm,
               sidxb, rows0, rows1, rows2, rows3, srow,
               sem0, sem1, sem2, sem3):
    c = lax.axis_index("c")
    s = lax.axis_index("s")
    wid = c * 16 + s
    pltpu.sync_copy(sidx.at[wid], sidxb)

    bufs = (rows0, rows1, rows2, rows3)
    sems = (sem0, sem1, sem2, sem3)

    def g_start(t, p):
        pltpu.async_copy(x_hbm.at[sidxb.at[t]], bufs[p], sems[p])

    def g_wait(p):
        pltpu.make_async_copy(x_hbm.at[sidxb.at[0]], bufs[p], sems[p]).wait()

    for t in range(4):
        g_start(t, t)
    for t in range(16):
        p = t % 4
        g_wait(p)
        buf = bufs[p]

        def quad(r, carry):
            out = list(carry)
            for q in range(4):
                for k in range(4):
                    out[k] = out[k] + buf[4 * r + q, pl.ds(16 * k, 16)]
                    out[4 + k] = out[4 + k] + buf[64 + 4 * r + q,
                                                  pl.ds(16 * k, 16)]
            return tuple(out)

        z = jnp.zeros((16,), jnp.float32)
        acc8 = lax.fori_loop(0, 16, quad, (z,) * 8)
        for k in range(4):
            srow[2 * t, pl.ds(16 * k, 16)] = acc8[k]
            srow[2 * t + 1, pl.ds(16 * k, 16)] = acc8[4 + k]
        if t + 4 < 16:
            g_start(t + 4, p)
    pltpu.sync_copy(srow, out_hbm.at[pl.ds(wid * BPW, BPW), :])


_sentence = pl.kernel(
    _sent_body,
    out_type=jax.ShapeDtypeStruct((B, F2), jnp.float32),
    mesh=_mesh,
    scratch_types=[
        pltpu.VMEM((16, 2 * LPAD), jnp.int32),
        pltpu.VMEM((2 * LPAD, F2), jnp.float32),
        pltpu.VMEM((2 * LPAD, F2), jnp.float32),
        pltpu.VMEM((2 * LPAD, F2), jnp.float32),
        pltpu.VMEM((2 * LPAD, F2), jnp.float32),
        pltpu.VMEM((BPW, F2), jnp.float32),
        pltpu.SemaphoreType.DMA,
        pltpu.SemaphoreType.DMA,
        pltpu.SemaphoreType.DMA,
        pltpu.SemaphoreType.DMA,
    ],
    compiler_params=pltpu.CompilerParams(use_tc_tiling_on_sc=False),
)


def _tc_mm_body(x_ref, w_ref, o_ref):
    o_ref[...] = jnp.dot(x_ref[...], w_ref[...],
                         preferred_element_type=jnp.float32)


def _tc_mid_body(p_ref, d_ref, b1_ref, w2_ref, o_ref):
    agg = p_ref[0] + p_ref[1]
    deg = d_ref[0] + d_ref[1]
    inv = 1.0 / jnp.maximum(deg, 1.0)
    h = jnp.maximum(agg * inv + b1_ref[...], 0.0)
    o_ref[...] = jnp.dot(h, w2_ref[...], preferred_element_type=jnp.float32)


def _tc_x_body(q_ref, d_ref, b2_ref, o_ref):
    agg = q_ref[0] + q_ref[1]
    deg = d_ref[0] + d_ref[1]
    inv = 1.0 / jnp.maximum(deg, 1.0)
    x = agg * inv + b2_ref[...]
    row = lax.broadcasted_iota(jnp.int32, (N, F2), 0)
    o_ref[...] = jnp.where(row == 0, 0.0, x)


def _tc_head_body(s_ref, w1_ref, c1_ref, w2_ref, c2_ref, w3_ref, c3_ref, o_ref):
    h = jnp.dot(s_ref[...], w1_ref[...], preferred_element_type=jnp.float32)
    h = jnp.maximum(h + c1_ref[...], 0.0)
    h = jnp.dot(h, w2_ref[...], preferred_element_type=jnp.float32)
    h = jnp.maximum(h + c2_ref[...], 0.0)
    o_ref[...] = jnp.dot(h, w3_ref[...],
                         preferred_element_type=jnp.float32) + c3_ref[...]


def kernel(sentence, edge_index, inputs, W1, b1, W2, b2,
           Wf1, bf1, Wf2, bf2, Wf3, bf3):
    ei = jnp.asarray(edge_index, jnp.int32)
    src5 = ei[0].reshape(NW, NP1, CPP1, K)
    dst5 = ei[1].reshape(NW, NP1, CPP1, K)
    src3 = ei[0].reshape(NW, 1, NCH, K)
    dst3 = ei[1].reshape(NW, 1, NCH, K)
    z128 = jnp.zeros((N, F1), jnp.float32)
    z64 = jnp.zeros((N, F2), jnp.float32)
    sent_pad = jnp.concatenate(
        [jnp.asarray(sentence, jnp.int32),
         jnp.zeros((B, LPAD - L), jnp.int32)], axis=1).reshape(NW, 16, 2 * LPAD)
    loc3 = jnp.asarray(_loc_np)

    y1 = pl.pallas_call(
        _tc_mm_body,
        out_shape=jax.ShapeDtypeStruct((N, F1), jnp.float32),
    )(inputs, W1)

    aggp, degp = _edge128(y1, src5, dst5, z128)
    degp = degp.reshape(2, N, 1)

    y2 = pl.pallas_call(
        _tc_mid_body,
        out_shape=jax.ShapeDtypeStruct((N, F2), jnp.float32),
    )(aggp, degp, b1.reshape(1, F1), W2)

    qp = _edge64(y2, src3, dst3, z64)

    x = pl.pallas_call(
        _tc_x_body,
        out_shape=jax.ShapeDtypeStruct((N, F2), jnp.float32),
    )(qp, degp, b2.reshape(1, F2))

    sent = _sentence(x, sent_pad)

    out = pl.pallas_call(
        _tc_head_body,
        out_shape=jax.ShapeDtypeStruct((B, 2), jnp.float32),
    )(sent, Wf1, bf1.reshape(1, 256), Wf2, bf2.reshape(1, 128),
      Wf3, bf3.reshape(1, 2))
    return out


# speedup vs baseline: 1.0104x; 1.0104x over previous
import functools

import numpy as np
import jax
import jax.numpy as jnp
from jax import lax
from jax.experimental import pallas as pl
from jax.experimental.pallas import tpu as pltpu
from jax.experimental.pallas import tpu_sc as plsc

N = 10000
E = 320000
F1 = 128
F2 = 64
B = 1024
L = 50
LPAD = 64
NW = 32
EPW = E // NW
K = 80
NCH = EPW // K
RPT = 624
BPW = B // NW

_mesh = plsc.VectorSubcoreMesh(core_axis_name="c", subcore_axis_name="s",
                               num_cores=2, num_subcores=16)

_loc_np = np.zeros((16, 16, 128), np.int32)
for _s in range(16):
    for _t in range(16):
        _loc_np[_s, _t, :64] = _s * BPW + 2 * _t
        _loc_np[_s, _t, 64:] = _s * BPW + 2 * _t + 1


def _striped(s, fn):
    fn(s * RPT, RPT)

    @pl.when(s == 15)
    def _tail():
        fn(16 * RPT, N - 16 * RPT)


def _edge_body(F, with_deg, n_phase, cpp, *refs):
    if with_deg:
        (tab, src4, dst4, z2d, agg_out, deg_out,
         srcb, dstb, rows0, rows1, ones, dbuf, acc, dacc, sem0, sem1) = refs
    else:
        (tab, src4, dst4, z2d, agg_out,
         srcb, dstb, rows0, rows1, acc, sem0, sem1) = refs
    c = lax.axis_index("c")
    s = lax.axis_index("s")
    wid = c * 16 + s
    _striped(s, lambda o, n: pltpu.sync_copy(z2d.at[pl.ds(o, n), :],
                                             acc.at[pl.ds(o, n), :]))
    if with_deg:
        for i in range(K // 16):
            ones[pl.ds(i * 16, 16)] = jnp.ones((16,), jnp.float32)
        for i in range(RPT // 16):
            dbuf[pl.ds(i * 16, 16)] = jnp.zeros((16,), jnp.float32)
        _striped(s, lambda o, n: pltpu.sync_copy(dbuf.at[pl.ds(0, n)],
                                                 dacc.at[pl.ds(o, n)]))
    plsc.subcore_barrier()

    def g_start(j, buf, sem):
        pltpu.async_copy(tab.at[srcb.at[j]], buf, sem)

    def g_wait(buf, sem):
        pltpu.make_async_copy(tab.at[srcb.at[0]], buf, sem).wait()

    def scat(j, buf):
        pltpu.sync_copy(buf, acc.at[dstb.at[j]], add=True)
        if with_deg:
            pltpu.sync_copy(ones, dacc.at[dstb.at[j]], add=True)

    for p in range(n_phase):
        pltpu.sync_copy(src4.at[wid, p], srcb)
        pltpu.sync_copy(dst4.at[wid, p], dstb)
        g_start(0, rows0, sem0)

        def chunk2(j2, carry):
            a = 2 * j2
            g_wait(rows0, sem0)
            g_start(a + 1, rows1, sem1)
            scat(a, rows0)
            g_wait(rows1, sem1)
            g_start(a + 2, rows0, sem0)
            scat(a + 1, rows1)
            return carry

        lax.fori_loop(0, (cpp - 1) // 2, chunk2, 0)
        g_wait(rows0, sem0)
        scat(cpp - 1, rows0)
    plsc.subcore_barrier()
    _striped(s, lambda o, n: pltpu.sync_copy(acc.at[pl.ds(o, n), :],
                                             agg_out.at[c, pl.ds(o, n), :]))
    if with_deg:
        def _deg_out(o, n):
            pltpu.sync_copy(dacc.at[pl.ds(o, n)], dbuf.at[pl.ds(0, n)])
            pltpu.sync_copy(dbuf.at[pl.ds(0, n)], deg_out.at[pl.ds(c * N + o, n)])
        _striped(s, _deg_out)


NP1 = 5
CPP1 = NCH // NP1

_edge128 = pl.kernel(
    functools.partial(_edge_body, F1, True, NP1, CPP1),
    out_type=(jax.ShapeDtypeStruct((2, N, F1), jnp.float32),
              jax.ShapeDtypeStruct((2 * N,), jnp.float32)),
    mesh=_mesh,
    scratch_types=[
        pltpu.VMEM((CPP1, K), jnp.int32),
        pltpu.VMEM((CPP1, K), jnp.int32),
        pltpu.VMEM((K, F1), jnp.float32),
        pltpu.VMEM((K, F1), jnp.float32),
        pltpu.VMEM((K,), jnp.float32),
        pltpu.VMEM((RPT,), jnp.float32),
        pltpu.VMEM_SHARED((N, F1), jnp.float32),
        pltpu.VMEM_SHARED((N,), jnp.float32),
        pltpu.SemaphoreType.DMA,
        pltpu.SemaphoreType.DMA,
    ],
)

_edge64 = pl.kernel(
    functools.partial(_edge_body, F2, False, 1, NCH),
    out_type=jax.ShapeDtypeStruct((2, N, F2), jnp.float32),
    mesh=_mesh,
    scratch_types=[
        pltpu.VMEM((NCH, K), jnp.int32),
        pltpu.VMEM((NCH, K), jnp.int32),
        pltpu.VMEM((K, F2), jnp.float32),
        pltpu.VMEM((K, F2), jnp.float32),
        pltpu.VMEM_SHARED((N, F2), jnp.float32),
        pltpu.SemaphoreType.DMA,
        pltpu.SemaphoreType.DMA,
    ],
    compiler_params=pltpu.CompilerParams(use_tc_tiling_on_sc=False),
)


def _sent_body(x_hbm, sidx, out_hbm,
               sidxb, rows0, rows1, rows2, rows3, srow,
               sem0, sem1, sem2, sem3):
    c = lax.axis_index("c")
    s = lax.axis_index("s")
    wid = c * 16 + s
    pltpu.sync_copy(sidx.at[wid], sidxb)

    bufs = (rows0, rows1, rows2, rows3)
    sems = (sem0, sem1, sem2, sem3)

    def g_start(t, p):
        pltpu.async_copy(x_hbm.at[sidxb.at[t]], bufs[p], sems[p])

    def g_wait(p):
        pltpu.make_async_copy(x_hbm.at[sidxb.at[0]], bufs[p], sems[p]).wait()

    for t in range(4):
        g_start(t, t)
    for t in range(16):
        p = t % 4
        g_wait(p)
        buf = bufs[p]

        def quad(r, carry):
            out = list(carry)
            for q in range(4):
                for k in range(4):
                    out[k] = out[k] + buf[4 * r + q, pl.ds(16 * k, 16)]
                    out[4 + k] = out[4 + k] + buf[64 + 4 * r + q,
                                                  pl.ds(16 * k, 16)]
            return tuple(out)

        z = jnp.zeros((16,), jnp.float32)
        acc8 = lax.fori_loop(0, 16, quad, (z,) * 8)
        for k in range(4):
            srow[2 * t, pl.ds(16 * k, 16)] = acc8[k]
            srow[2 * t + 1, pl.ds(16 * k, 16)] = acc8[4 + k]
        if t + 4 < 16:
            g_start(t + 4, p)
    pltpu.sync_copy(srow, out_hbm.at[pl.ds(wid * BPW, BPW), :])


_sentence = pl.kernel(
    _sent_body,
    out_type=jax.ShapeDtypeStruct((B, F2), jnp.float32),
    mesh=_mesh,
    scratch_types=[
        pltpu.VMEM((16, 2 * LPAD), jnp.int32),
        pltpu.VMEM((2 * LPAD, F2), jnp.float32),
        pltpu.VMEM((2 * LPAD, F2), jnp.float32),
        pltpu.VMEM((2 * LPAD, F2), jnp.float32),
        pltpu.VMEM((2 * LPAD, F2), jnp.float32),
        pltpu.VMEM((BPW, F2), jnp.float32),
        pltpu.SemaphoreType.DMA,
        pltpu.SemaphoreType.DMA,
        pltpu.SemaphoreType.DMA,
        pltpu.SemaphoreType.DMA,
    ],
    compiler_params=pltpu.CompilerParams(use_tc_tiling_on_sc=False),
)


def _tc_mid_body(p_ref, d_ref, w1_ref, b1_ref, w2_ref, o_ref):
    agg = p_ref[0] + p_ref[1]
    deg = d_ref[0] + d_ref[1]
    inv = 1.0 / jnp.maximum(deg, 1.0)
    h1 = jnp.dot(agg * inv, w1_ref[...], preferred_element_type=jnp.float32)
    h = jnp.maximum(h1 + b1_ref[...], 0.0)
    o_ref[...] = jnp.dot(h, w2_ref[...], preferred_element_type=jnp.float32)


def _tc_x_body(q_ref, d_ref, b2_ref, o_ref):
    agg = q_ref[0] + q_ref[1]
    deg = d_ref[0] + d_ref[1]
    inv = 1.0 / jnp.maximum(deg, 1.0)
    x = agg * inv + b2_ref[...]
    row = lax.broadcasted_iota(jnp.int32, (N, F2), 0)
    o_ref[...] = jnp.where(row == 0, 0.0, x)


def _tc_head_body(s_ref, w1_ref, c1_ref, w2_ref, c2_ref, w3_ref, c3_ref, o_ref):
    h = jnp.dot(s_ref[...], w1_ref[...], preferred_element_type=jnp.float32)
    h = jnp.maximum(h + c1_ref[...], 0.0)
    h = jnp.dot(h, w2_ref[...], preferred_element_type=jnp.float32)
    h = jnp.maximum(h + c2_ref[...], 0.0)
    o_ref[...] = jnp.dot(h, w3_ref[...],
                         preferred_element_type=jnp.float32) + c3_ref[...]


def kernel(sentence, edge_index, inputs, W1, b1, W2, b2,
           Wf1, bf1, Wf2, bf2, Wf3, bf3):
    ei = jnp.asarray(edge_index, jnp.int32)
    src5 = ei[0].reshape(NW, NP1, CPP1, K)
    dst5 = ei[1].reshape(NW, NP1, CPP1, K)
    src3 = ei[0].reshape(NW, 1, NCH, K)
    dst3 = ei[1].reshape(NW, 1, NCH, K)
    z128 = jnp.zeros((N, F1), jnp.float32)
    z64 = jnp.zeros((N, F2), jnp.float32)
    sent_pad = jnp.concatenate(
        [jnp.asarray(sentence, jnp.int32),
         jnp.zeros((B, LPAD - L), jnp.int32)], axis=1).reshape(NW, 16, 2 * LPAD)
    loc3 = jnp.asarray(_loc_np)

    aggp, degp = _edge128(inputs, src5, dst5, z128)
    degp = degp.reshape(2, N, 1)

    y2 = pl.pallas_call(
        _tc_mid_body,
        out_shape=jax.ShapeDtypeStruct((N, F2), jnp.float32),
    )(aggp, degp, W1, b1.reshape(1, F1), W2)

    qp = _edge64(y2, src3, dst3, z64)

    x = pl.pallas_call(
        _tc_x_body,
        out_shape=jax.ShapeDtypeStruct((N, F2), jnp.float32),
    )(qp, degp, b2.reshape(1, F2))

    sent = _sentence(x, sent_pad)

    out = pl.pallas_call(
        _tc_head_body,
        out_shape=jax.ShapeDtypeStruct((B, 2), jnp.float32),
    )(sent, Wf1, bf1.reshape(1, 256), Wf2, bf2.reshape(1, 128),
      Wf3, bf3.reshape(1, 2))
    return out
